# baseline (speedup 1.0000x reference)
import functools

import jax
import jax.numpy as jnp
from jax import lax
from jax.experimental import pallas as pl
from jax.experimental.pallas import tpu as pltpu
from jax.experimental.pallas import tpu_sc as plsc

B, T, N, C, K = 8, 100, 196, 256, 16
ROWS = B * K
D = N * C
NCORES, NSUB = 2, 16
NW = NCORES * NSUB
RPW = ROWS // NW

_mesh = plsc.VectorSubcoreMesh(
    core_axis_name="c", subcore_axis_name="s",
    num_cores=NCORES, num_subcores=NSUB)


@functools.partial(
    pl.kernel,
    out_type=(
        jax.ShapeDtypeStruct((ROWS, D), jnp.float32),
        jax.ShapeDtypeStruct((ROWS, C), jnp.float32),
    ),
    mesh=_mesh,
    scratch_types=[
        pltpu.VMEM((NW, RPW, 1), jnp.int32),
        pltpu.VMEM((NW * 8,), jnp.int32),
        pltpu.VMEM((1, D), jnp.float32),
        pltpu.VMEM((1, D), jnp.float32),
        pltpu.VMEM((RPW, C), jnp.float32),
        pltpu.SemaphoreType.DMA,
        pltpu.SemaphoreType.DMA,
        pltpu.SemaphoreType.DMA,
        pltpu.SemaphoreType.DMA,
        pltpu.SemaphoreType.DMA,
    ],
)
def _sc_gather(flat_hbm, flat1d_hbm, patch_hbm, audio_hbm, outp_hbm, outa_hbm,
               idx_v, aidx, buf0, buf1, abuf, sg0, sg1, sw0, sw1, sa):
    wid = lax.axis_index("s") * NCORES + lax.axis_index("c")
    base = wid * RPW

    pltpu.sync_copy(flat_hbm, idx_v)
    pltpu.sync_copy(flat1d_hbm, aidx)

    bufs = (buf0, buf1)
    sgs = (sg0, sg1)
    sws = (sw0, sw1)

    ah = pltpu.async_copy(
        audio_hbm.at[aidx.at[pl.ds(wid * 8, RPW)]], abuf, sa)

    gh = [None] * RPW
    wh = [None] * RPW
    for r in range(2):
        gh[r] = pltpu.async_copy(
            patch_hbm.at[idx_v.at[wid, r]], bufs[r], sgs[r])
    for r in range(RPW):
        bsel = r % 2
        gh[r].wait()
        wh[r] = pltpu.async_copy(
            bufs[bsel], outp_hbm.at[pl.ds(base + r, 1)], sws[bsel])
        if r + 2 < RPW:
            wh[r].wait()
            gh[r + 2] = pltpu.async_copy(
                patch_hbm.at[idx_v.at[wid, r + 2]], bufs[bsel], sgs[bsel])
    wh[RPW - 2].wait()
    wh[RPW - 1].wait()

    ah.wait()
    pltpu.sync_copy(abuf, outa_hbm.at[pl.ds(base, RPW)])


def kernel(top_k_index_sort, patch_feat, audio_feat):
    idx = top_k_index_sort[:, 0, :].astype(jnp.int32)
    flat = (jnp.arange(B, dtype=jnp.int32)[:, None] * T + idx)
    flat_pad = jnp.concatenate(
        [flat.reshape(NW, RPW),
         jnp.zeros((NW, 8 - RPW), jnp.int32)], axis=1).reshape(NW * 8)
    outp, outa = _sc_gather(
        flat.reshape(NW, RPW, 1),
        flat_pad,
        patch_feat.reshape(B * T, D),
        audio_feat.reshape(B * T, C),
    )
    return outp.reshape(B, K, N, C), outa.reshape(B, K, C)
